# Initial kernel scaffold; baseline (speedup 1.0000x reference)
import jax
import jax.numpy as jnp
from jax.experimental import pallas as pl


def kernel(x, edge_index, W1, b1, Wc1, bc1, Wc3, bc3):
    raise NotImplementedError("write your pallas kernel here")


# speedup vs baseline: 7.0354x; 7.0354x over previous
import functools

import jax
import jax.numpy as jnp
from jax import lax
from jax.experimental import pallas as pl
from jax.experimental.pallas import tpu as pltpu
from jax.experimental.pallas import tpu_sc as plsc

N = 10000
D = 128
H = 128
E = 320000

NC = 2
NS = 16
NW = NC * NS
PER_TILE = E // NW
CHUNK = 128
EP = 10240
NCHUNK = EP // CHUNK
NPASS = 2
CPP = NCHUNK // NPASS
NP = 10240
RPS = NP // NS

_MESH = plsc.VectorSubcoreMesh(core_axis_name="c", subcore_axis_name="s")


def _deg_body(edges_hbm, e01_hbm, zeros128_hbm, deg_hbm,
              src_v, dst_v, e0_v, e1_v, deg_sh):
    c = lax.axis_index("c")
    s = lax.axis_index("s")
    wid = s * NC + c
    pltpu.sync_copy(e01_hbm.at[0], e0_v)
    pltpu.sync_copy(e01_hbm.at[1], e1_v)
    r0 = s * RPS
    pltpu.sync_copy(zeros128_hbm.at[pl.ds(r0, RPS)], deg_sh.at[pl.ds(r0, RPS)])
    plsc.subcore_barrier()

    def pass_body(p, carry):
        pltpu.sync_copy(edges_hbm.at[0, wid, p], src_v)
        pltpu.sync_copy(edges_hbm.at[1, wid, p], dst_v)

        def body(j, carry2):
            pltpu.sync_copy(e0_v, deg_sh.at[src_v.at[j]], add=True)
            pltpu.sync_copy(e1_v, deg_sh.at[dst_v.at[j]], add=True)
            return carry2

        lax.fori_loop(0, CPP, body, 0)
        return carry

    lax.fori_loop(0, NPASS, pass_body, 0)
    plsc.subcore_barrier()
    pltpu.sync_copy(deg_sh.at[pl.ds(r0, RPS)], deg_hbm.at[c, pl.ds(r0, RPS)])


_deg_call = functools.partial(
    pl.kernel,
    out_type=jax.ShapeDtypeStruct((NC, NP, H), jnp.float32),
    mesh=_MESH,
    scratch_types=[
        pltpu.VMEM((CPP, CHUNK), jnp.int32),
        pltpu.VMEM((CPP, CHUNK), jnp.int32),
        pltpu.VMEM((CHUNK, H), jnp.float32),
        pltpu.VMEM((CHUNK, H), jnp.float32),
        pltpu.VMEM_SHARED((NP, H), jnp.float32),
    ],
)(_deg_body)


def _agg_body(hs_hbm, edges_hbm, zeros128_hbm, out_hbm,
              src_v, dst_v, rows0, rows1, gsem0, gsem1, agg_sh):
    c = lax.axis_index("c")
    s = lax.axis_index("s")
    wid = s * NC + c
    r0 = s * RPS
    pltpu.sync_copy(zeros128_hbm.at[pl.ds(r0, RPS)], agg_sh.at[pl.ds(r0, RPS)])
    plsc.subcore_barrier()

    def pass_body(p, carry):
        pltpu.sync_copy(edges_hbm.at[0, wid, p], src_v)
        pltpu.sync_copy(edges_hbm.at[1, wid, p], dst_v)
        pltpu.async_copy(hs_hbm.at[src_v.at[0]], rows0, gsem0)

        def body(i, carry2):
            j = 2 * i
            pltpu.async_copy(hs_hbm.at[src_v.at[j + 1]], rows1, gsem1)
            pltpu.make_async_copy(hs_hbm.at[src_v.at[0]], rows0, gsem0).wait()
            pltpu.sync_copy(rows0, agg_sh.at[dst_v.at[j]], add=True)

            @pl.when(j + 2 < CPP)
            def _():
                pltpu.async_copy(hs_hbm.at[src_v.at[j + 2]], rows0, gsem0)

            pltpu.make_async_copy(hs_hbm.at[src_v.at[0]], rows1, gsem1).wait()
            pltpu.sync_copy(rows1, agg_sh.at[dst_v.at[j + 1]], add=True)
            return carry2

        lax.fori_loop(0, CPP // 2, body, 0)
        return carry

    lax.fori_loop(0, NPASS, pass_body, 0)
    plsc.subcore_barrier()
    pltpu.sync_copy(agg_sh.at[pl.ds(r0, RPS)], out_hbm.at[c, pl.ds(r0, RPS)])


_agg_call = functools.partial(
    pl.kernel,
    out_type=jax.ShapeDtypeStruct((NC, NP, H), jnp.float32),
    mesh=_MESH,
    scratch_types=[
        pltpu.VMEM((CPP, CHUNK), jnp.int32),
        pltpu.VMEM((CPP, CHUNK), jnp.int32),
        pltpu.VMEM((CHUNK, H), jnp.float32),
        pltpu.VMEM((CHUNK, H), jnp.float32),
        pltpu.SemaphoreType.DMA,
        pltpu.SemaphoreType.DMA,
        pltpu.VMEM_SHARED((NP, H), jnp.float32),
    ],
)(_agg_body)


def _mm_body(x_ref, w_ref, d0_ref, d1_ref, out_ref):
    deg = jnp.maximum(d0_ref[...] + d1_ref[...], 1.0)
    h = jnp.dot(x_ref[...], w_ref[...], preferred_element_type=jnp.float32)
    out_ref[...] = h * lax.rsqrt(deg)


def _mm_call(x, W1, d0, d1):
    return pl.pallas_call(
        _mm_body,
        out_shape=jax.ShapeDtypeStruct((NP, H), jnp.float32),
    )(x, W1, d0, d1)


def _head_body(a0_ref, a1_ref, d0_ref, d1_ref, b1_ref, wc1_ref, bc1_ref,
               wc3_ref, bc3_ref, out_ref):
    deg = jnp.maximum(d0_ref[...] + d1_ref[...], 1.0)
    conv = (a0_ref[...] + a1_ref[...]) * lax.rsqrt(deg) + b1_ref[...]
    h1 = jnp.maximum(conv, 0.0)
    hg = jnp.sum(h1, axis=0, keepdims=True) * (1.0 / N)
    t = jnp.dot(hg, wc1_ref[...], preferred_element_type=jnp.float32) + bc1_ref[...]
    t = jnp.maximum(t, 0.0)
    out_ref[...] = jnp.dot(t, wc3_ref[...], preferred_element_type=jnp.float32) + bc3_ref[...]


def _head_call(a0, a1, d0, d1, b1, Wc1, bc1, Wc3, bc3):
    return pl.pallas_call(
        _head_body,
        out_shape=jax.ShapeDtypeStruct((1, 1), jnp.float32),
    )(a0, a1, d0, d1, b1, Wc1, bc1, Wc3, bc3)


def kernel(x, edge_index, W1, b1, Wc1, bc1, Wc3, bc3):
    e = edge_index.astype(jnp.int32).reshape(2, NW, PER_TILE)
    pad = jnp.full((2, NW, EP - PER_TILE), N, jnp.int32)
    e_agg = jnp.concatenate([e, pad], axis=2).reshape(2, NW, NPASS, CPP, CHUNK)
    e01 = jnp.zeros((2, CHUNK, H), jnp.float32)
    e01 = e01.at[0, :, 0].set(1.0).at[1, :, 1].set(1.0)
    zeros128 = jnp.zeros((NP, H), jnp.float32)
    x_p = jnp.zeros((NP, D), jnp.float32).at[:N].set(x)

    deg = _deg_call(e_agg, e01, zeros128)
    hs = _mm_call(x_p, W1, deg[0, :, 0:1], deg[1, :, 0:1])
    aggp = _agg_call(hs, e_agg, zeros128)
    out = _head_call(aggp[0, :N], aggp[1, :N], deg[0, :N, 1:2], deg[1, :N, 1:2],
                     b1.reshape(1, H), Wc1, bc1.reshape(1, H), Wc3,
                     bc3.reshape(1, 1))
    return out.reshape(1)


# speedup vs baseline: 7.0467x; 1.0016x over previous
import functools

import jax
import jax.numpy as jnp
from jax import lax
from jax.experimental import pallas as pl
from jax.experimental.pallas import tpu as pltpu
from jax.experimental.pallas import tpu_sc as plsc

N = 10000
D = 128
H = 128
E = 320000

NC = 2
NS = 16
NW = NC * NS
PER_TILE = E // NW
CHUNK = 128
EP = 10240
NCHUNK = EP // CHUNK
NPASS = 2
CPP = NCHUNK // NPASS
ACHUNK = 64
ANCHUNK = EP // ACHUNK
ANPASS = 4
ACPP = ANCHUNK // ANPASS
NP = 10240
RPS = NP // NS

_MESH = plsc.VectorSubcoreMesh(core_axis_name="c", subcore_axis_name="s")


def _deg_body(edges_hbm, e01_hbm, zeros128_hbm, deg_hbm,
              src_v, dst_v, e0_v, e1_v, deg_sh):
    c = lax.axis_index("c")
    s = lax.axis_index("s")
    wid = s * NC + c
    pltpu.sync_copy(e01_hbm.at[0], e0_v)
    pltpu.sync_copy(e01_hbm.at[1], e1_v)
    r0 = s * RPS
    pltpu.sync_copy(zeros128_hbm.at[pl.ds(r0, RPS)], deg_sh.at[pl.ds(r0, RPS)])
    plsc.subcore_barrier()

    def pass_body(p, carry):
        pltpu.sync_copy(edges_hbm.at[0, wid, p], src_v)
        pltpu.sync_copy(edges_hbm.at[1, wid, p], dst_v)

        def body(j, carry2):
            pltpu.sync_copy(e0_v, deg_sh.at[src_v.at[j]], add=True)
            pltpu.sync_copy(e1_v, deg_sh.at[dst_v.at[j]], add=True)
            return carry2

        lax.fori_loop(0, CPP, body, 0)
        return carry

    lax.fori_loop(0, NPASS, pass_body, 0)
    plsc.subcore_barrier()
    pltpu.sync_copy(deg_sh.at[pl.ds(r0, RPS)], deg_hbm.at[c, pl.ds(r0, RPS)])


_deg_call = functools.partial(
    pl.kernel,
    out_type=jax.ShapeDtypeStruct((NC, NP, H), jnp.float32),
    mesh=_MESH,
    scratch_types=[
        pltpu.VMEM((CPP, CHUNK), jnp.int32),
        pltpu.VMEM((CPP, CHUNK), jnp.int32),
        pltpu.VMEM((CHUNK, H), jnp.float32),
        pltpu.VMEM((CHUNK, H), jnp.float32),
        pltpu.VMEM_SHARED((NP, H), jnp.float32),
    ],
)(_deg_body)


def _agg_body(hs_hbm, edges_hbm, zeros128_hbm, out_hbm,
              src_v, dst_v, rows0, rows1, rows2, rows3,
              sem0, sem1, sem2, sem3, agg_sh):
    c = lax.axis_index("c")
    s = lax.axis_index("s")
    wid = s * NC + c
    r0 = s * RPS
    pltpu.sync_copy(zeros128_hbm.at[pl.ds(r0, RPS)], agg_sh.at[pl.ds(r0, RPS)])
    plsc.subcore_barrier()

    bufs = [(rows0, sem0), (rows1, sem1), (rows2, sem2), (rows3, sem3)]
    NBUF = len(bufs)

    def gather(j, b):
        pltpu.async_copy(hs_hbm.at[src_v.at[j]], bufs[b][0], bufs[b][1])

    def drain_scatter(j, b):
        pltpu.make_async_copy(hs_hbm.at[src_v.at[0]], bufs[b][0],
                              bufs[b][1]).wait()
        pltpu.sync_copy(bufs[b][0], agg_sh.at[dst_v.at[j]], add=True)

    def pass_body(p, carry):
        pltpu.sync_copy(edges_hbm.at[0, wid, p], src_v)
        pltpu.sync_copy(edges_hbm.at[1, wid, p], dst_v)
        for b in range(NBUF - 1):
            gather(b, b)

        def body(k, carry2):
            j = NBUF * k
            gather(j + NBUF - 1, NBUF - 1)
            for b in range(NBUF):
                drain_scatter(j + b, b)
                if b < NBUF - 1:
                    @pl.when(j + NBUF + b < ACPP)
                    def _():
                        gather(j + NBUF + b, b)
            return carry2

        lax.fori_loop(0, ACPP // NBUF, body, 0)
        return carry

    lax.fori_loop(0, ANPASS, pass_body, 0)
    plsc.subcore_barrier()
    pltpu.sync_copy(agg_sh.at[pl.ds(r0, RPS)], out_hbm.at[c, pl.ds(r0, RPS)])


_agg_call = functools.partial(
    pl.kernel,
    out_type=jax.ShapeDtypeStruct((NC, NP, H), jnp.float32),
    mesh=_MESH,
    scratch_types=[
        pltpu.VMEM((ACPP, ACHUNK), jnp.int32),
        pltpu.VMEM((ACPP, ACHUNK), jnp.int32),
        pltpu.VMEM((ACHUNK, H), jnp.float32),
        pltpu.VMEM((ACHUNK, H), jnp.float32),
        pltpu.VMEM((ACHUNK, H), jnp.float32),
        pltpu.VMEM((ACHUNK, H), jnp.float32),
        pltpu.SemaphoreType.DMA,
        pltpu.SemaphoreType.DMA,
        pltpu.SemaphoreType.DMA,
        pltpu.SemaphoreType.DMA,
        pltpu.VMEM_SHARED((NP, H), jnp.float32),
    ],
)(_agg_body)


def _mm_body(x_ref, w_ref, d0_ref, d1_ref, out_ref):
    deg = jnp.maximum(d0_ref[...] + d1_ref[...], 1.0)
    h = jnp.dot(x_ref[...], w_ref[...], preferred_element_type=jnp.float32)
    out_ref[...] = h * lax.rsqrt(deg)


def _mm_call(x, W1, d0, d1):
    return pl.pallas_call(
        _mm_body,
        out_shape=jax.ShapeDtypeStruct((NP, H), jnp.float32),
    )(x, W1, d0, d1)


def _head_body(a0_ref, a1_ref, d0_ref, d1_ref, b1_ref, wc1_ref, bc1_ref,
               wc3_ref, bc3_ref, out_ref):
    deg = jnp.maximum(d0_ref[...] + d1_ref[...], 1.0)
    conv = (a0_ref[...] + a1_ref[...]) * lax.rsqrt(deg) + b1_ref[...]
    h1 = jnp.maximum(conv, 0.0)
    hg = jnp.sum(h1, axis=0, keepdims=True) * (1.0 / N)
    t = jnp.dot(hg, wc1_ref[...], preferred_element_type=jnp.float32) + bc1_ref[...]
    t = jnp.maximum(t, 0.0)
    out_ref[...] = jnp.dot(t, wc3_ref[...], preferred_element_type=jnp.float32) + bc3_ref[...]


def _head_call(a0, a1, d0, d1, b1, Wc1, bc1, Wc3, bc3):
    return pl.pallas_call(
        _head_body,
        out_shape=jax.ShapeDtypeStruct((1, 1), jnp.float32),
    )(a0, a1, d0, d1, b1, Wc1, bc1, Wc3, bc3)


def kernel(x, edge_index, W1, b1, Wc1, bc1, Wc3, bc3):
    e = edge_index.astype(jnp.int32).reshape(2, NW, PER_TILE)
    pad = jnp.full((2, NW, EP - PER_TILE), N, jnp.int32)
    ep = jnp.concatenate([e, pad], axis=2)
    e_deg = ep.reshape(2, NW, NPASS, CPP, CHUNK)
    e_agg = ep.reshape(2, NW, ANPASS, ACPP, ACHUNK)
    e01 = jnp.zeros((2, CHUNK, H), jnp.float32)
    e01 = e01.at[0, :, 0].set(1.0).at[1, :, 1].set(1.0)
    zeros128 = jnp.zeros((NP, H), jnp.float32)
    x_p = jnp.zeros((NP, D), jnp.float32).at[:N].set(x)

    deg = _deg_call(e_deg, e01, zeros128)
    hs = _mm_call(x_p, W1, deg[0, :, 0:1], deg[1, :, 0:1])
    aggp = _agg_call(hs, e_agg, zeros128)
    out = _head_call(aggp[0, :N], aggp[1, :N], deg[0, :N, 1:2], deg[1, :N, 1:2],
                     b1.reshape(1, H), Wc1, bc1.reshape(1, H), Wc3,
                     bc3.reshape(1, 1))
    return out.reshape(1)
